# baseline (speedup 1.0000x reference)
import functools

import jax
import jax.numpy as jnp
from jax import lax
from jax.experimental import pallas as pl
from jax.experimental.pallas import tpu as pltpu
from jax.experimental.pallas import tpu_sc as plsc

B = 4096
L = 200
EMB = 64
NC = 2
NS = 16
NW = NC * NS
ROWS_PER_W = B // NW
CH0 = 104
CH1 = L - CH0
NBUF = 4
_POOL_PERM = (list(range(0, 32, 2)) + list(range(1, 32, 2))
              + list(range(32, 64, 2)) + list(range(33, 64, 2)))


def _pooled_sums(emb_table, idx_padded):
    mesh = plsc.VectorSubcoreMesh(core_axis_name="c", subcore_axis_name="s")

    @functools.partial(
        pl.kernel,
        out_type=jax.ShapeDtypeStruct((B, EMB), jnp.float32),
        mesh=mesh,
        compiler_params=pltpu.CompilerParams(use_tc_tiling_on_sc=False,
                                             needs_layout_passes=False),
        scratch_types=[
            pltpu.VMEM((ROWS_PER_W, L), jnp.int32),
            pltpu.VMEM((NBUF, L, EMB), jnp.bfloat16),
            pltpu.VMEM((ROWS_PER_W, EMB), jnp.float32),
        ] + [pltpu.SemaphoreType.DMA] * NBUF,
    )
    def sc_kernel(table_hbm, idx_hbm, out_hbm, idx_v, rows_v, out_v, *sems):
        wid = lax.axis_index("s") * NC + lax.axis_index("c")
        base = wid * ROWS_PER_W
        pltpu.sync_copy(idx_hbm.at[pl.ds(base, ROWS_PER_W)], idx_v)

        def start(r, slot):
            pltpu.async_copy(
                table_hbm.at[idx_v.at[r, pl.ds(0, CH0)]],
                rows_v.at[slot, pl.ds(0, CH0)], sems[slot])
            pltpu.async_copy(
                table_hbm.at[idx_v.at[r, pl.ds(CH0, CH1)]],
                rows_v.at[slot, pl.ds(CH0, CH1)], sems[slot])

        def wait(slot):
            pltpu.make_async_copy(
                table_hbm.at[pl.ds(0, L)], rows_v.at[slot],
                sems[slot]).wait()

        def accum(r, slot):
            def body(i, acc):
                new = []
                for h in range(2):
                    v = rows_v[slot, i, pl.ds(h * 32, 32)]
                    u = plsc.bitcast(v, jnp.uint32)
                    lo = plsc.bitcast(u << 16, jnp.float32)
                    hi = plsc.bitcast(u & jnp.uint32(0xFFFF0000),
                                      jnp.float32)
                    new.append(acc[2 * h] + lo)
                    new.append(acc[2 * h + 1] + hi)
                return tuple(new)

            z = jnp.zeros((16,), jnp.float32)
            a = lax.fori_loop(0, L, body, (z, z, z, z), unroll=8)
            for c in range(4):
                out_v[r, pl.ds(c * 16, 16)] = a[c]

        for b in range(NBUF):
            start(b, b)

        @pl.loop(0, ROWS_PER_W, step=NBUF)
        def _(r):
            for b in range(NBUF):
                wait(b)
                accum(r + b, b)

                @pl.when(r + b + NBUF < ROWS_PER_W)
                def _():
                    start(r + b + NBUF, b)

        pltpu.sync_copy(out_v, out_hbm.at[pl.ds(base, ROWS_PER_W)])

    return sc_kernel(emb_table, idx_padded)


def _bn(x, gamma, beta, eps=1e-5):
    mean = jnp.mean(x, axis=0)
    var = jnp.var(x, axis=0)
    return gamma * (x - mean) / jnp.sqrt(var + eps) + beta


def _dense_body(ts_ref, pooled_ref, cat_ref, catW_ref, catb_ref, bg_ref,
                bb_ref, W1a_ref, W1b_ref, b1_ref, g1_ref, be1_ref, W2_ref,
                b2_ref, g2_ref, be2_ref, W3_ref, b3_ref, out_ref):
    hp = lax.Precision.HIGHEST
    cnt = jnp.sum((ts_ref[...] != 0).astype(jnp.float32), axis=1,
                  keepdims=True)
    text_feat = pooled_ref[...] / (cnt + 1e-9)
    cat = jnp.dot(cat_ref[...], catW_ref[...], precision=hp) + catb_ref[...]
    cat = jax.nn.relu(_bn(cat, bg_ref[...], bb_ref[...]))
    h = (jnp.dot(text_feat, W1a_ref[...], precision=hp)
         + jnp.dot(cat, W1b_ref[...], precision=hp) + b1_ref[...])
    h = jax.nn.relu(_bn(h, g1_ref[...], be1_ref[...]))
    h = jnp.dot(h, W2_ref[...], precision=hp) + b2_ref[...]
    h = jax.nn.relu(_bn(h, g2_ref[...], be2_ref[...]))
    out_ref[...] = jnp.dot(h, W3_ref[...], precision=hp) + b3_ref[...]


def kernel(text_seq, cat_features, emb_table, cat_W, cat_b, bn_cat_g,
           bn_cat_b, W1, b1, g1, be1, W2, b2, g2, be2, W3, b3):
    idx = text_seq.astype(jnp.int32)
    pooled = _pooled_sums(emb_table.astype(jnp.bfloat16), idx)

    row = lambda v: v.reshape(1, -1)
    out = pl.pallas_call(
        _dense_body,
        out_shape=jax.ShapeDtypeStruct((B, 1), jnp.float32),
    )(idx, pooled, cat_features, cat_W, row(cat_b), row(bn_cat_g),
      row(bn_cat_b), W1[:EMB][jnp.array(_POOL_PERM)], W1[EMB:], row(b1), row(g1),
      row(be1), W2, row(b2), row(g2), row(be2), W3, row(b3))
    return out


# speedup vs baseline: 1.0949x; 1.0949x over previous
import functools

import jax
import jax.numpy as jnp
from jax import lax
from jax.experimental import pallas as pl
from jax.experimental.pallas import tpu as pltpu
from jax.experimental.pallas import tpu_sc as plsc

B = 4096
L = 200
EMB = 64
NC = 2
NS = 16
NW = NC * NS
ROWS_PER_W = B // NW
CH0 = 104
CH1 = L - CH0
NBUF = 4


def _pooled_sums(emb_table, idx):
    mesh = plsc.VectorSubcoreMesh(core_axis_name="c", subcore_axis_name="s")

    @functools.partial(
        pl.kernel,
        out_type=jax.ShapeDtypeStruct((B, EMB), jnp.float32),
        mesh=mesh,
        compiler_params=pltpu.CompilerParams(use_tc_tiling_on_sc=False),
        scratch_types=[
            pltpu.VMEM((ROWS_PER_W, L), jnp.int32),
            pltpu.VMEM((NBUF, L, EMB), jnp.float32),
            pltpu.VMEM((ROWS_PER_W, EMB), jnp.float32),
        ] + [pltpu.SemaphoreType.DMA] * NBUF,
    )
    def sc_kernel(table_hbm, idx_hbm, out_hbm, idx_v, rows_v, out_v, *sems):
        wid = lax.axis_index("s") * NC + lax.axis_index("c")
        base = wid * ROWS_PER_W
        pltpu.sync_copy(idx_hbm.at[pl.ds(base, ROWS_PER_W)], idx_v)

        def start(r, slot):
            pltpu.async_copy(
                table_hbm.at[idx_v.at[r, pl.ds(0, CH0)]],
                rows_v.at[slot, pl.ds(0, CH0)], sems[slot])
            pltpu.async_copy(
                table_hbm.at[idx_v.at[r, pl.ds(CH0, CH1)]],
                rows_v.at[slot, pl.ds(CH0, CH1)], sems[slot])

        def wait(slot):
            pltpu.make_async_copy(
                table_hbm.at[pl.ds(0, L)], rows_v.at[slot],
                sems[slot]).wait()

        def accum(r, slot):
            def body(i, acc):
                return tuple(
                    acc[c] + rows_v[slot, i, pl.ds(c * 16, 16)]
                    for c in range(4))

            z = jnp.zeros((16,), jnp.float32)
            a = lax.fori_loop(0, L, body, (z, z, z, z), unroll=8)
            for c in range(4):
                out_v[r, pl.ds(c * 16, 16)] = a[c]

        for b in range(NBUF):
            start(b, b)

        @pl.loop(0, ROWS_PER_W, step=NBUF)
        def _(r):
            for b in range(NBUF):
                wait(b)
                accum(r + b, b)

                @pl.when(r + b + NBUF < ROWS_PER_W)
                def _():
                    start(r + b + NBUF, b)

        pltpu.sync_copy(out_v, out_hbm.at[pl.ds(base, ROWS_PER_W)])

    return sc_kernel(emb_table, idx)


def _bn(x, gamma, beta, eps=1e-5):
    n = x.shape[0]
    s1 = jnp.sum(x, axis=0)
    s2 = jnp.sum(x * x, axis=0)
    mean = s1 * (1.0 / n)
    var = s2 * (1.0 / n) - mean * mean
    return gamma * (x - mean) / jnp.sqrt(var + eps) + beta


def _cat_body(ts_ref, cat_ref, catW_ref, catb_ref, bg_ref, bb_ref, W1b_ref,
              b1_ref, hcat_ref, cnt_ref):
    hp = lax.Precision.HIGHEST
    cnt_ref[...] = jnp.sum((ts_ref[...] != 0).astype(jnp.float32), axis=1,
                           keepdims=True)
    cat = jnp.dot(cat_ref[...], catW_ref[...], precision=hp) + catb_ref[...]
    cat = jax.nn.relu(_bn(cat, bg_ref[...], bb_ref[...]))
    hcat_ref[...] = (jnp.dot(cat, W1b_ref[...], precision=hp) + b1_ref[...])


def _final_body(pooled_ref, hcat_ref, cnt_ref, W1a_ref, g1_ref, be1_ref,
                W2_ref, b2_ref, g2_ref, be2_ref, W3_ref, b3_ref, out_ref):
    hp = lax.Precision.HIGHEST
    text_feat = pooled_ref[...] / (cnt_ref[...] + 1e-9)
    h = jnp.dot(text_feat, W1a_ref[...], precision=hp) + hcat_ref[...]
    h = jax.nn.relu(_bn(h, g1_ref[...], be1_ref[...]))
    h = jnp.dot(h, W2_ref[...], precision=hp) + b2_ref[...]
    h = jax.nn.relu(_bn(h, g2_ref[...], be2_ref[...]))
    out_ref[...] = jnp.dot(h, W3_ref[...], precision=hp) + b3_ref[...]


def kernel(text_seq, cat_features, emb_table, cat_W, cat_b, bn_cat_g,
           bn_cat_b, W1, b1, g1, be1, W2, b2, g2, be2, W3, b3):
    idx = text_seq.astype(jnp.int32)
    pooled = _pooled_sums(emb_table, idx)

    row = lambda v: v.reshape(1, -1)
    hcat, cnt = pl.pallas_call(
        _cat_body,
        out_shape=(jax.ShapeDtypeStruct((B, 256), jnp.float32),
                   jax.ShapeDtypeStruct((B, 1), jnp.float32)),
    )(idx, cat_features, cat_W, row(cat_b), row(bn_cat_g), row(bn_cat_b),
      W1[EMB:], row(b1))

    out = pl.pallas_call(
        _final_body,
        out_shape=jax.ShapeDtypeStruct((B, 1), jnp.float32),
    )(pooled, hcat, cnt, W1[:EMB], row(g1), row(be1), W2, row(b2), row(g2),
      row(be2), W3, row(b3))
    return out


# speedup vs baseline: 1.0960x; 1.0010x over previous
import functools

import jax
import jax.numpy as jnp
from jax import lax
from jax.experimental import pallas as pl
from jax.experimental.pallas import tpu as pltpu
from jax.experimental.pallas import tpu_sc as plsc

B = 4096
L = 200
EMB = 64
NC = 2
NS = 16
NW = NC * NS
ROWS_PER_W = B // NW
CH0 = 104
CH1 = L - CH0
NBUF = 4


def _pooled_sums(emb_table, idx):
    mesh = plsc.VectorSubcoreMesh(core_axis_name="c", subcore_axis_name="s")

    @functools.partial(
        pl.kernel,
        out_type=jax.ShapeDtypeStruct((B, EMB), jnp.float32),
        mesh=mesh,
        compiler_params=pltpu.CompilerParams(use_tc_tiling_on_sc=False),
        scratch_types=[
            pltpu.VMEM((ROWS_PER_W, L), jnp.int32),
            pltpu.VMEM((NBUF, L, EMB), jnp.float32),
            pltpu.VMEM((ROWS_PER_W, EMB), jnp.float32),
        ] + [pltpu.SemaphoreType.DMA] * NBUF,
    )
    def sc_kernel(table_hbm, idx_hbm, out_hbm, idx_v, rows_v, out_v, *sems):
        wid = lax.axis_index("s") * NC + lax.axis_index("c")
        base = wid * ROWS_PER_W
        pltpu.sync_copy(idx_hbm.at[pl.ds(base, ROWS_PER_W)], idx_v)

        def start(r, slot):
            pltpu.async_copy(
                table_hbm.at[idx_v.at[r]], rows_v.at[slot], sems[slot])

        def wait(slot):
            pltpu.make_async_copy(
                table_hbm.at[pl.ds(0, L)], rows_v.at[slot],
                sems[slot]).wait()

        def accum(r, slot):
            def body(i, acc):
                return tuple(
                    acc[c] + rows_v[slot, i, pl.ds(c * 16, 16)]
                    for c in range(4))

            z = jnp.zeros((16,), jnp.float32)
            a = lax.fori_loop(0, L, body, (z, z, z, z), unroll=8)
            for c in range(4):
                out_v[r, pl.ds(c * 16, 16)] = a[c]

        for b in range(NBUF):
            start(b, b)

        @pl.loop(0, ROWS_PER_W, step=NBUF)
        def _(r):
            for b in range(NBUF):
                wait(b)
                accum(r + b, b)

                @pl.when(r + b + NBUF < ROWS_PER_W)
                def _():
                    start(r + b + NBUF, b)

        pltpu.sync_copy(out_v, out_hbm.at[pl.ds(base, ROWS_PER_W)])

    return sc_kernel(emb_table, idx)


def _bn(x, gamma, beta, eps=1e-5):
    n = x.shape[0]
    s1 = jnp.sum(x, axis=0)
    s2 = jnp.sum(x * x, axis=0)
    mean = s1 * (1.0 / n)
    var = s2 * (1.0 / n) - mean * mean
    return gamma * (x - mean) / jnp.sqrt(var + eps) + beta


def _cat_body(ts_ref, cat_ref, catW_ref, catb_ref, bg_ref, bb_ref, W1b_ref,
              b1_ref, hcat_ref, cnt_ref):
    hp = lax.Precision.HIGHEST
    cnt_ref[...] = jnp.sum((ts_ref[...] != 0).astype(jnp.float32), axis=1,
                           keepdims=True)
    cat = jnp.dot(cat_ref[...], catW_ref[...], precision=hp) + catb_ref[...]
    cat = jax.nn.relu(_bn(cat, bg_ref[...], bb_ref[...]))
    hcat_ref[...] = (jnp.dot(cat, W1b_ref[...], precision=hp) + b1_ref[...])


def _final_body(pooled_ref, hcat_ref, cnt_ref, W1a_ref, g1_ref, be1_ref,
                W2_ref, b2_ref, g2_ref, be2_ref, W3_ref, b3_ref, out_ref):
    hp = lax.Precision.HIGHEST
    text_feat = pooled_ref[...] / (cnt_ref[...] + 1e-9)
    h = jnp.dot(text_feat, W1a_ref[...], precision=hp) + hcat_ref[...]
    h = jax.nn.relu(_bn(h, g1_ref[...], be1_ref[...]))
    h = jnp.dot(h, W2_ref[...], precision=hp) + b2_ref[...]
    h = jax.nn.relu(_bn(h, g2_ref[...], be2_ref[...]))
    out_ref[...] = jnp.dot(h, W3_ref[...], precision=hp) + b3_ref[...]


def kernel(text_seq, cat_features, emb_table, cat_W, cat_b, bn_cat_g,
           bn_cat_b, W1, b1, g1, be1, W2, b2, g2, be2, W3, b3):
    idx = text_seq.astype(jnp.int32)
    pooled = _pooled_sums(emb_table, idx)

    row = lambda v: v.reshape(1, -1)
    hcat, cnt = pl.pallas_call(
        _cat_body,
        out_shape=(jax.ShapeDtypeStruct((B, 256), jnp.float32),
                   jax.ShapeDtypeStruct((B, 1), jnp.float32)),
    )(idx, cat_features, cat_W, row(cat_b), row(bn_cat_g), row(bn_cat_b),
      W1[EMB:], row(b1))

    out = pl.pallas_call(
        _final_body,
        out_shape=jax.ShapeDtypeStruct((B, 1), jnp.float32),
    )(pooled, hcat, cnt, W1[:EMB], row(g1), row(be1), W2, row(b2), row(g2),
      row(be2), W3, row(b3))
    return out


# speedup vs baseline: 1.0980x; 1.0019x over previous
import functools

import jax
import jax.numpy as jnp
from jax import lax
from jax.experimental import pallas as pl
from jax.experimental.pallas import tpu as pltpu
from jax.experimental.pallas import tpu_sc as plsc

B = 4096
L = 200
EMB = 64
NC = 2
NS = 16
NW = NC * NS
ROWS_PER_W = B // NW
CH0 = 104
CH1 = L - CH0
NBUF = 4


def _pooled_sums(emb_table, idx):
    mesh = plsc.VectorSubcoreMesh(core_axis_name="c", subcore_axis_name="s")

    @functools.partial(
        pl.kernel,
        out_type=jax.ShapeDtypeStruct((B, EMB), jnp.float32),
        mesh=mesh,
        compiler_params=pltpu.CompilerParams(use_tc_tiling_on_sc=False),
        scratch_types=[
            pltpu.VMEM((ROWS_PER_W, L), jnp.int32),
            pltpu.VMEM((NBUF, L, EMB), jnp.float32),
            pltpu.VMEM((ROWS_PER_W, EMB), jnp.float32),
        ] + [pltpu.SemaphoreType.DMA] * NBUF,
    )
    def sc_kernel(table_hbm, idx_hbm, out_hbm, idx_v, rows_v, out_v, *sems):
        wid = lax.axis_index("s") * NC + lax.axis_index("c")
        base = wid * ROWS_PER_W
        pltpu.sync_copy(idx_hbm.at[pl.ds(base, ROWS_PER_W)], idx_v)

        def start(r, slot):
            pltpu.async_copy(
                table_hbm.at[idx_v.at[r, pl.ds(0, CH0)]],
                rows_v.at[slot, pl.ds(0, CH0)], sems[slot])
            pltpu.async_copy(
                table_hbm.at[idx_v.at[r, pl.ds(CH0, CH1)]],
                rows_v.at[slot, pl.ds(CH0, CH1)], sems[slot])

        def wait(slot):
            pltpu.make_async_copy(
                table_hbm.at[pl.ds(0, L)], rows_v.at[slot],
                sems[slot]).wait()

        def accum(r, slot):
            def body(i, acc):
                return tuple(
                    acc[c] + rows_v[slot, i, pl.ds(c * 16, 16)]
                    for c in range(4))

            z = jnp.zeros((16,), jnp.float32)
            a = lax.fori_loop(0, L, body, (z, z, z, z), unroll=8)
            for c in range(4):
                out_v[r, pl.ds(c * 16, 16)] = a[c]

        for b in range(NBUF):
            start(b, b)

        @pl.loop(0, ROWS_PER_W, step=NBUF)
        def _(r):
            for b in range(NBUF):
                wait(b)
                accum(r + b, b)

                @pl.when(r + b + NBUF < ROWS_PER_W)
                def _():
                    start(r + b + NBUF, b)

        pltpu.sync_copy(out_v, out_hbm.at[pl.ds(base, ROWS_PER_W)])

    return sc_kernel(emb_table, idx)


def _bn(x, gamma, beta, eps=1e-5):
    n = x.shape[0]
    s1 = jnp.sum(x, axis=0)
    s2 = jnp.sum(x * x, axis=0)
    mean = s1 * (1.0 / n)
    var = s2 * (1.0 / n) - mean * mean
    return gamma * (x - mean) / jnp.sqrt(var + eps) + beta


def _cat_body(ts_ref, cat_ref, catW_ref, catb_ref, bg_ref, bb_ref, W1b_ref,
              b1_ref, hcat_ref, cnt_ref):
    hp = lax.Precision.HIGHEST
    cnt_ref[...] = jnp.sum((ts_ref[...] != 0).astype(jnp.float32), axis=1,
                           keepdims=True)
    cat = jnp.dot(cat_ref[...], catW_ref[...], precision=hp) + catb_ref[...]
    cat = jax.nn.relu(_bn(cat, bg_ref[...], bb_ref[...]))
    hcat_ref[...] = (jnp.dot(cat, W1b_ref[...], precision=hp) + b1_ref[...])


def _final_body(pooled_ref, hcat_ref, cnt_ref, W1a_ref, g1_ref, be1_ref,
                W2_ref, b2_ref, g2_ref, be2_ref, W3_ref, b3_ref, out_ref):
    hp = lax.Precision.HIGHEST
    text_feat = pooled_ref[...] / (cnt_ref[...] + 1e-9)
    h = jnp.dot(text_feat, W1a_ref[...], precision=hp) + hcat_ref[...]
    h = jax.nn.relu(_bn(h, g1_ref[...], be1_ref[...]))
    h = jnp.dot(h, W2_ref[...], precision=hp) + b2_ref[...]
    h = jax.nn.relu(_bn(h, g2_ref[...], be2_ref[...]))
    out_ref[...] = jnp.dot(h, W3_ref[...], precision=hp) + b3_ref[...]


def kernel(text_seq, cat_features, emb_table, cat_W, cat_b, bn_cat_g,
           bn_cat_b, W1, b1, g1, be1, W2, b2, g2, be2, W3, b3):
    idx = text_seq.astype(jnp.int32)
    pooled = _pooled_sums(emb_table, idx)

    row = lambda v: v.reshape(1, -1)
    hcat, cnt = pl.pallas_call(
        _cat_body,
        out_shape=(jax.ShapeDtypeStruct((B, 256), jnp.float32),
                   jax.ShapeDtypeStruct((B, 1), jnp.float32)),
    )(idx, cat_features, cat_W, row(cat_b), row(bn_cat_g), row(bn_cat_b),
      W1[EMB:], row(b1))

    out = pl.pallas_call(
        _final_body,
        out_shape=jax.ShapeDtypeStruct((B, 1), jnp.float32),
    )(pooled, hcat, cnt, W1[:EMB], row(g1), row(be1), W2, row(b2), row(g2),
      row(be2), W3, row(b3))
    return out
